# initial kernel scaffold (unmeasured)
import jax
import jax.numpy as jnp
from jax import lax
from jax.experimental import pallas as pl
from jax.experimental.pallas import tpu as pltpu

N_DEV = 8

_printed = False


def _debug_topology():
    global _printed
    if _printed:
        return
    _printed = True
    try:
        for d in jax.devices():
            print(
                "[topo]", d.id, getattr(d, "coords", None),
                getattr(d, "core_on_chip", None),
            )
    except Exception as e:
        print("[topo] failed:", e)


def kernel(x, w_mat):
    _debug_topology()
    m_per, k = x.shape
    kw, n = w_mat.shape
    assert kw == k
    n_per = n // N_DEV

    def body(x_ref, w_hbm, out_ref, wblk_ref, send_ref,
             copy_sems, send_sems, recv_sems):
        my = lax.axis_index("i")

        barrier = pltpu.get_barrier_semaphore()
        for d in range(N_DEV):
            pl.semaphore_signal(
                barrier, inc=1, device_id=(d,),
                device_id_type=pl.DeviceIdType.MESH,
            )
        pl.semaphore_wait(barrier, N_DEV)

        def wcopy(s):
            j = (my + s) % N_DEV
            return pltpu.make_async_copy(
                w_hbm.at[:, pl.ds(j * n_per, n_per)],
                wblk_ref.at[s % 2],
                copy_sems.at[s % 2],
            )

        copies = [wcopy(0)]
        copies[0].start()
        rdmas = []
        for s in range(N_DEV):
            if s + 1 < N_DEV:
                copies.append(wcopy(s + 1))
                copies[s + 1].start()
            copies[s].wait()
            z = jnp.maximum(
                jnp.dot(x_ref[...], wblk_ref[s % 2],
                        preferred_element_type=jnp.float32),
                0.0,
            )
            if s == 0:
                out_ref[pl.ds(my * m_per, m_per), :] = z
            else:
                j = (my + s) % N_DEV
                send_ref[s - 1] = z
                rdma = pltpu.make_async_remote_copy(
                    src_ref=send_ref.at[s - 1],
                    dst_ref=out_ref.at[pl.ds(my * m_per, m_per), :],
                    send_sem=send_sems.at[s - 1],
                    recv_sem=recv_sems.at[s - 1],
                    device_id=(j,),
                    device_id_type=pl.DeviceIdType.MESH,
                )
                rdma.start()
                rdmas.append(rdma)

        for s in range(1, N_DEV):
            rdmas[s - 1].wait_send()
            src_p = (my - s) % N_DEV
            recv = pltpu.make_async_remote_copy(
                src_ref=send_ref.at[s - 1],
                dst_ref=out_ref.at[pl.ds(src_p * m_per, m_per), :],
                send_sem=send_sems.at[s - 1],
                recv_sem=recv_sems.at[s - 1],
                device_id=(my,),
                device_id_type=pl.DeviceIdType.MESH,
            )
            recv.wait_recv()

    return pl.pallas_call(
        body,
        out_shape=jax.ShapeDtypeStruct((N_DEV * m_per, n_per), jnp.float32),
        in_specs=[
            pl.BlockSpec(memory_space=pltpu.VMEM),
            pl.BlockSpec(memory_space=pltpu.ANY),
        ],
        out_specs=pl.BlockSpec(memory_space=pltpu.VMEM),
        scratch_shapes=[
            pltpu.VMEM((2, k, n_per), jnp.float32),
            pltpu.VMEM((N_DEV - 1, m_per, n_per), jnp.float32),
            pltpu.SemaphoreType.DMA((2,)),
            pltpu.SemaphoreType.DMA((N_DEV - 1,)),
            pltpu.SemaphoreType.DMA((N_DEV - 1,)),
        ],
        compiler_params=pltpu.CompilerParams(collective_id=0),
    )(x, w_mat)


# baseline (device time: 172557 ns/iter reference)
import jax
import jax.numpy as jnp
from jax import lax
from jax.experimental import pallas as pl
from jax.experimental.pallas import tpu as pltpu

N_DEV = 8

_printed = False


def _debug_topology():
    global _printed
    if _printed:
        return
    _printed = True
    try:
        for d in jax.devices():
            print(
                "[topo]", d.id, getattr(d, "coords", None),
                getattr(d, "core_on_chip", None),
            )
    except Exception as e:
        print("[topo] failed:", e)


def kernel(x, w_mat):
    _debug_topology()
    m_per, k = x.shape
    kw, n = w_mat.shape
    assert kw == k
    n_per = n // N_DEV

    def body(x_ref, w_hbm, out_ref, wblk_ref, send_ref,
             copy_sems, send_sems, recv_sems):
        my = lax.axis_index("i")

        barrier = pltpu.get_barrier_semaphore()
        for d in range(N_DEV):
            pl.semaphore_signal(
                barrier, inc=1, device_id=(d,),
                device_id_type=pl.DeviceIdType.MESH,
            )
        pl.semaphore_wait(barrier, N_DEV)

        n_half = n_per // 2
        n_sub = 2 * N_DEV

        def wcopy(u):
            s, t = u // 2, u % 2
            j = (my + s) % N_DEV
            return pltpu.make_async_copy(
                w_hbm.at[:, pl.ds(j * n_per + t * n_half, n_half)],
                wblk_ref.at[u % 2],
                copy_sems.at[u % 2],
            )

        copies = [wcopy(0)]
        copies[0].start()
        rdmas = []
        for u in range(n_sub):
            s, t = u // 2, u % 2
            if u + 1 < n_sub:
                copies.append(wcopy(u + 1))
                copies[u + 1].start()
            copies[u].wait()
            z = jnp.maximum(
                jnp.dot(x_ref[...], wblk_ref[u % 2],
                        preferred_element_type=jnp.float32),
                0.0,
            )
            cols = slice(t * n_half, (t + 1) * n_half)
            if s == 0:
                out_ref[pl.ds(my * m_per, m_per), cols] = z
            else:
                send_ref[s - 1, :, cols] = z
                if t == 1:
                    j = (my + s) % N_DEV
                    rdma = pltpu.make_async_remote_copy(
                        src_ref=send_ref.at[s - 1],
                        dst_ref=out_ref.at[pl.ds(my * m_per, m_per), :],
                        send_sem=send_sems.at[s - 1],
                        recv_sem=recv_sems.at[s - 1],
                        device_id=(j,),
                        device_id_type=pl.DeviceIdType.MESH,
                    )
                    rdma.start()
                    rdmas.append(rdma)

        for s in range(1, N_DEV):
            rdmas[s - 1].wait_send()
            src_p = (my - s) % N_DEV
            recv = pltpu.make_async_remote_copy(
                src_ref=send_ref.at[s - 1],
                dst_ref=out_ref.at[pl.ds(src_p * m_per, m_per), :],
                send_sem=send_sems.at[s - 1],
                recv_sem=recv_sems.at[s - 1],
                device_id=(my,),
                device_id_type=pl.DeviceIdType.MESH,
            )
            recv.wait_recv()

    return pl.pallas_call(
        body,
        out_shape=jax.ShapeDtypeStruct((N_DEV * m_per, n_per), jnp.float32),
        in_specs=[
            pl.BlockSpec(memory_space=pltpu.VMEM),
            pl.BlockSpec(memory_space=pl.ANY),
        ],
        out_specs=pl.BlockSpec(memory_space=pltpu.VMEM),
        scratch_shapes=[
            pltpu.VMEM((2, k, n_per // 2), jnp.float32),
            pltpu.VMEM((N_DEV - 1, m_per, n_per), jnp.float32),
            pltpu.SemaphoreType.DMA((2,)),
            pltpu.SemaphoreType.DMA((N_DEV - 1,)),
            pltpu.SemaphoreType.DMA((N_DEV - 1,)),
        ],
        compiler_params=pltpu.CompilerParams(
            collective_id=0, vmem_limit_bytes=100 * 1024 * 1024,
        ),
    )(x, w_mat)


# device time: 169183 ns/iter; 1.0199x vs baseline; 1.0199x over previous
import os

import jax
import jax.numpy as jnp
from jax import lax
from jax.experimental import pallas as pl
from jax.experimental.pallas import tpu as pltpu

N_DEV = 8
_ABLATE = os.environ.get("SCB_ABLATE", "")

_printed = False


def _debug_topology():
    global _printed
    if _printed:
        return
    _printed = True
    try:
        for d in jax.devices():
            print(
                "[topo]", d.id, getattr(d, "coords", None),
                getattr(d, "core_on_chip", None),
            )
    except Exception as e:
        print("[topo] failed:", e)


def kernel(x, w_mat):
    _debug_topology()
    m_per, k = x.shape
    kw, n = w_mat.shape
    assert kw == k
    n_per = n // N_DEV

    def body(x_ref, w_hbm, out_ref, wblk_ref, send_ref,
             copy_sems, send_sems, recv_sems):
        my = lax.axis_index("i")

        barrier = pltpu.get_barrier_semaphore()
        for d in range(N_DEV):
            pl.semaphore_signal(
                barrier, inc=1, device_id=(d,),
                device_id_type=pl.DeviceIdType.MESH,
            )
        pl.semaphore_wait(barrier, N_DEV)

        n_half = n_per // 2
        n_sub = 2 * N_DEV

        def wcopy(u):
            s, t = u // 2, u % 2
            j = (my + s) % N_DEV
            return pltpu.make_async_copy(
                w_hbm.at[:, pl.ds(j * n_per + t * n_half, n_half)],
                wblk_ref.at[u % 2],
                copy_sems.at[u % 2],
            )

        copies = [wcopy(0)]
        copies[0].start()
        rdmas = []
        for u in range(n_sub):
            s, t = u // 2, u % 2
            if u + 1 < n_sub:
                copies.append(wcopy(u + 1))
                copies[u + 1].start()
            copies[u].wait()
            if _ABLATE != "nocompute":
                z = jnp.maximum(
                    jnp.dot(x_ref[...], wblk_ref[u % 2],
                            preferred_element_type=jnp.float32),
                    0.0,
                )
            else:
                z = wblk_ref[u % 2, : m_per, :]
            cols = slice(t * n_half, (t + 1) * n_half)
            if s == 0:
                out_ref[pl.ds(my * m_per, m_per), cols] = z
            else:
                send_ref[s - 1, :, cols] = z
                if t == 1 and _ABLATE != "nocomm":
                    j = (my + s) % N_DEV
                    rdma = pltpu.make_async_remote_copy(
                        src_ref=send_ref.at[s - 1],
                        dst_ref=out_ref.at[pl.ds(my * m_per, m_per), :],
                        send_sem=send_sems.at[s - 1],
                        recv_sem=recv_sems.at[s - 1],
                        device_id=(j,),
                        device_id_type=pl.DeviceIdType.MESH,
                    )
                    rdma.start()
                    rdmas.append(rdma)

        if _ABLATE != "nocomm":
            for s in range(1, N_DEV):
                rdmas[s - 1].wait_send()
                src_p = (my - s) % N_DEV
                recv = pltpu.make_async_remote_copy(
                    src_ref=send_ref.at[s - 1],
                    dst_ref=out_ref.at[pl.ds(src_p * m_per, m_per), :],
                    send_sem=send_sems.at[s - 1],
                    recv_sem=recv_sems.at[s - 1],
                    device_id=(my,),
                    device_id_type=pl.DeviceIdType.MESH,
                )
                recv.wait_recv()

    return pl.pallas_call(
        body,
        out_shape=jax.ShapeDtypeStruct((N_DEV * m_per, n_per), jnp.float32),
        in_specs=[
            pl.BlockSpec(memory_space=pltpu.VMEM),
            pl.BlockSpec(memory_space=pl.ANY),
        ],
        out_specs=pl.BlockSpec(memory_space=pltpu.VMEM),
        scratch_shapes=[
            pltpu.VMEM((2, k, n_per // 2), jnp.float32),
            pltpu.VMEM((N_DEV - 1, m_per, n_per), jnp.float32),
            pltpu.SemaphoreType.DMA((2,)),
            pltpu.SemaphoreType.DMA((N_DEV - 1,)),
            pltpu.SemaphoreType.DMA((N_DEV - 1,)),
        ],
        compiler_params=pltpu.CompilerParams(
            collective_id=0, vmem_limit_bytes=100 * 1024 * 1024,
        ),
    )(x, w_mat)


# device time: 104104 ns/iter; 1.6575x vs baseline; 1.6251x over previous
import os

import jax
import jax.numpy as jnp
from jax import lax
from jax.experimental import pallas as pl
from jax.experimental.pallas import tpu as pltpu

N_DEV = 8
_ABLATE = os.environ.get("SCB_ABLATE", "")

_printed = False


def _debug_topology():
    global _printed
    if _printed:
        return
    _printed = True
    try:
        for d in jax.devices():
            print(
                "[topo]", d.id, getattr(d, "coords", None),
                getattr(d, "core_on_chip", None),
            )
    except Exception as e:
        print("[topo] failed:", e)


def kernel(x, w_mat):
    _debug_topology()
    m_per, k = x.shape
    kw, n = w_mat.shape
    assert kw == k
    n_per = n // N_DEV

    def body(x_ref, w_hbm, out_ref, wblk_ref, send_ref, recv_ref,
             copy_sems, send_sems, recv_sems):
        my = lax.axis_index("i")

        barrier = pltpu.get_barrier_semaphore()
        for d in range(N_DEV):
            pl.semaphore_signal(
                barrier, inc=1, device_id=(d,),
                device_id_type=pl.DeviceIdType.MESH,
            )
        pl.semaphore_wait(barrier, N_DEV)

        n_half = n_per // 2
        n_sub = 2 * N_DEV

        def wcopy(u):
            s, t = u // 2, u % 2
            j = (my + s) % N_DEV
            return pltpu.make_async_copy(
                w_hbm.at[:, pl.ds(j * n_per + t * n_half, n_half)],
                wblk_ref.at[u % 2],
                copy_sems.at[u % 2],
            )

        copies = [wcopy(0)]
        copies[0].start()
        rdmas = []
        for u in range(n_sub):
            s, t = u // 2, u % 2
            if u + 1 < n_sub:
                copies.append(wcopy(u + 1))
                copies[u + 1].start()
            copies[u].wait()
            if _ABLATE != "nocompute":
                z = jnp.maximum(
                    jnp.dot(x_ref[...], wblk_ref[u % 2],
                            preferred_element_type=jnp.float32),
                    0.0,
                )
            else:
                z = wblk_ref[u % 2, : m_per, :]
            cols = slice(t * n_half, (t + 1) * n_half)
            if s == 0:
                out_ref[pl.ds(my * m_per, m_per), cols] = z
            else:
                send_ref[s - 1, :, cols] = z.astype(jnp.bfloat16)
                if t == 1 and _ABLATE != "nocomm":
                    j = (my + s) % N_DEV
                    rdma = pltpu.make_async_remote_copy(
                        src_ref=send_ref.at[s - 1],
                        dst_ref=recv_ref.at[s - 1],
                        send_sem=send_sems.at[s - 1],
                        recv_sem=recv_sems.at[s - 1],
                        device_id=(j,),
                        device_id_type=pl.DeviceIdType.MESH,
                    )
                    rdma.start()
                    rdmas.append(rdma)

        if _ABLATE != "nocomm":
            for s in range(1, N_DEV):
                rdmas[s - 1].wait_send()
                src_p = (my - s) % N_DEV
                recv = pltpu.make_async_remote_copy(
                    src_ref=send_ref.at[s - 1],
                    dst_ref=recv_ref.at[s - 1],
                    send_sem=send_sems.at[s - 1],
                    recv_sem=recv_sems.at[s - 1],
                    device_id=(my,),
                    device_id_type=pl.DeviceIdType.MESH,
                )
                recv.wait_recv()
                out_ref[pl.ds(src_p * m_per, m_per), :] = (
                    recv_ref[s - 1].astype(jnp.float32)
                )

    return pl.pallas_call(
        body,
        out_shape=jax.ShapeDtypeStruct((N_DEV * m_per, n_per), jnp.float32),
        in_specs=[
            pl.BlockSpec(memory_space=pltpu.VMEM),
            pl.BlockSpec(memory_space=pl.ANY),
        ],
        out_specs=pl.BlockSpec(memory_space=pltpu.VMEM),
        scratch_shapes=[
            pltpu.VMEM((2, k, n_per // 2), jnp.float32),
            pltpu.VMEM((N_DEV - 1, m_per, n_per), jnp.bfloat16),
            pltpu.VMEM((N_DEV - 1, m_per, n_per), jnp.bfloat16),
            pltpu.SemaphoreType.DMA((2,)),
            pltpu.SemaphoreType.DMA((N_DEV - 1,)),
            pltpu.SemaphoreType.DMA((N_DEV - 1,)),
        ],
        compiler_params=pltpu.CompilerParams(
            collective_id=0, vmem_limit_bytes=100 * 1024 * 1024,
        ),
    )(x, w_mat)


# device time: 95869 ns/iter; 1.7999x vs baseline; 1.0859x over previous
import os

import jax
import jax.numpy as jnp
from jax import lax
from jax.experimental import pallas as pl
from jax.experimental.pallas import tpu as pltpu

N_DEV = 8
_ABLATE = os.environ.get("SCB_ABLATE", "")

_printed = False


def _debug_topology():
    global _printed
    if _printed:
        return
    _printed = True
    try:
        for d in jax.devices():
            print(
                "[topo]", d.id, getattr(d, "coords", None),
                getattr(d, "core_on_chip", None),
            )
    except Exception as e:
        print("[topo] failed:", e)


def kernel(x, w_mat):
    _debug_topology()
    m_per, k = x.shape
    kw, n = w_mat.shape
    assert kw == k
    n_per = n // N_DEV

    def body(x_ref, w_hbm, out_ref, wblk_ref, send_ref, recv_ref,
             copy_sems, send_sems, recv_sems):
        my = lax.axis_index("i")

        barrier = pltpu.get_barrier_semaphore()
        for d in range(N_DEV):
            pl.semaphore_signal(
                barrier, inc=1, device_id=(d,),
                device_id_type=pl.DeviceIdType.MESH,
            )
        pl.semaphore_wait(barrier, N_DEV)

        n_half = n_per // 2
        n_sub = 2 * N_DEV

        def step_of(u):
            s = u // 2 + 1
            return 0 if s == N_DEV else s

        def wcopy(u):
            s, t = step_of(u), u % 2
            j = (my + s) % N_DEV
            return pltpu.make_async_copy(
                w_hbm.at[:, pl.ds(j * n_per + t * n_half, n_half)],
                wblk_ref.at[u % 2],
                copy_sems.at[u % 2],
            )

        copies = [wcopy(0)]
        copies[0].start()
        rdmas = []
        for u in range(n_sub):
            s, t = step_of(u), u % 2
            if u + 1 < n_sub:
                copies.append(wcopy(u + 1))
                copies[u + 1].start()
            copies[u].wait()
            if _ABLATE != "nocompute":
                z = jnp.maximum(
                    jnp.dot(x_ref[...], wblk_ref[u % 2],
                            preferred_element_type=jnp.float32),
                    0.0,
                )
            else:
                z = wblk_ref[u % 2, : m_per, :]
            cols = slice(t * n_half, (t + 1) * n_half)
            if s == 0:
                out_ref[pl.ds(my * m_per, m_per), cols] = z
            else:
                v = 2 * (s - 1) + t
                send_ref[v] = z.astype(jnp.bfloat16)
                if _ABLATE != "nocomm":
                    j = (my + s) % N_DEV
                    rdma = pltpu.make_async_remote_copy(
                        src_ref=send_ref.at[v],
                        dst_ref=recv_ref.at[v],
                        send_sem=send_sems.at[v],
                        recv_sem=recv_sems.at[v],
                        device_id=(j,),
                        device_id_type=pl.DeviceIdType.MESH,
                    )
                    rdma.start()
                    rdmas.append(rdma)

        if _ABLATE != "nocomm":
            for v in range(2 * (N_DEV - 1)):
                s, t = v // 2 + 1, v % 2
                rdmas[v].wait_send()
                src_p = (my - s) % N_DEV
                recv = pltpu.make_async_remote_copy(
                    src_ref=send_ref.at[v],
                    dst_ref=recv_ref.at[v],
                    send_sem=send_sems.at[v],
                    recv_sem=recv_sems.at[v],
                    device_id=(my,),
                    device_id_type=pl.DeviceIdType.MESH,
                )
                recv.wait_recv()
                out_ref[
                    pl.ds(src_p * m_per, m_per),
                    t * n_half : (t + 1) * n_half,
                ] = recv_ref[v].astype(jnp.float32)

    return pl.pallas_call(
        body,
        out_shape=jax.ShapeDtypeStruct((N_DEV * m_per, n_per), jnp.float32),
        in_specs=[
            pl.BlockSpec(memory_space=pltpu.VMEM),
            pl.BlockSpec(memory_space=pl.ANY),
        ],
        out_specs=pl.BlockSpec(memory_space=pltpu.VMEM),
        scratch_shapes=[
            pltpu.VMEM((2, k, n_per // 2), jnp.float32),
            pltpu.VMEM((2 * (N_DEV - 1), m_per, n_per // 2), jnp.bfloat16),
            pltpu.VMEM((2 * (N_DEV - 1), m_per, n_per // 2), jnp.bfloat16),
            pltpu.SemaphoreType.DMA((2,)),
            pltpu.SemaphoreType.DMA((2 * (N_DEV - 1),)),
            pltpu.SemaphoreType.DMA((2 * (N_DEV - 1),)),
        ],
        compiler_params=pltpu.CompilerParams(
            collective_id=0, vmem_limit_bytes=100 * 1024 * 1024,
        ),
    )(x, w_mat)


# device time: 95495 ns/iter; 1.8070x vs baseline; 1.0039x over previous
import jax
import jax.numpy as jnp
from jax import lax
from jax.experimental import pallas as pl
from jax.experimental.pallas import tpu as pltpu

N_DEV = 8


def kernel(x, w_mat):
    m_per, k = x.shape
    kw, n = w_mat.shape
    assert kw == k
    n_per = n // N_DEV

    def body(x_ref, w_hbm, out_ref, wblk_ref, send_ref, recv_ref,
             copy_sems, send_sems, recv_sems):
        my = lax.axis_index("i")

        barrier = pltpu.get_barrier_semaphore()
        for d in range(N_DEV):
            pl.semaphore_signal(
                barrier, inc=1, device_id=(d,),
                device_id_type=pl.DeviceIdType.MESH,
            )
        pl.semaphore_wait(barrier, N_DEV)

        n_half = n_per // 2
        n_sub = 2 * N_DEV

        def step_of(u):
            s = u // 2 + 1
            return 0 if s == N_DEV else s

        def wcopy(u):
            s, t = step_of(u), u % 2
            j = (my + s) % N_DEV
            return pltpu.make_async_copy(
                w_hbm.at[:, pl.ds(j * n_per + t * n_half, n_half)],
                wblk_ref.at[u % 2],
                copy_sems.at[u % 2],
            )

        copies = [wcopy(0)]
        copies[0].start()
        rdmas = []
        for u in range(n_sub):
            s, t = step_of(u), u % 2
            if u + 1 < n_sub:
                copies.append(wcopy(u + 1))
                copies[u + 1].start()
            copies[u].wait()
            z = jnp.maximum(
                jnp.dot(x_ref[...], wblk_ref[u % 2],
                        preferred_element_type=jnp.float32),
                0.0,
            )
            cols = slice(t * n_half, (t + 1) * n_half)
            if s == 0:
                out_ref[pl.ds(my * m_per, m_per), cols] = z
            else:
                v = 2 * (s - 1) + t
                send_ref[v] = z.astype(jnp.bfloat16)
                j = (my + s) % N_DEV
                rdma = pltpu.make_async_remote_copy(
                    src_ref=send_ref.at[v],
                    dst_ref=recv_ref.at[v],
                    send_sem=send_sems.at[v],
                    recv_sem=recv_sems.at[v],
                    device_id=(j,),
                    device_id_type=pl.DeviceIdType.MESH,
                )
                rdma.start()
                rdmas.append(rdma)

        for v in range(2 * (N_DEV - 1)):
            s, t = v // 2 + 1, v % 2
            rdmas[v].wait_send()
            src_p = (my - s) % N_DEV
            recv = pltpu.make_async_remote_copy(
                src_ref=send_ref.at[v],
                dst_ref=recv_ref.at[v],
                send_sem=send_sems.at[v],
                recv_sem=recv_sems.at[v],
                device_id=(my,),
                device_id_type=pl.DeviceIdType.MESH,
            )
            recv.wait_recv()
            out_ref[
                pl.ds(src_p * m_per, m_per),
                t * n_half : (t + 1) * n_half,
            ] = recv_ref[v].astype(jnp.float32)

    return pl.pallas_call(
        body,
        out_shape=jax.ShapeDtypeStruct((N_DEV * m_per, n_per), jnp.float32),
        in_specs=[
            pl.BlockSpec(memory_space=pltpu.VMEM),
            pl.BlockSpec(memory_space=pl.ANY),
        ],
        out_specs=pl.BlockSpec(memory_space=pltpu.VMEM),
        scratch_shapes=[
            pltpu.VMEM((2, k, n_per // 2), jnp.float32),
            pltpu.VMEM((2 * (N_DEV - 1), m_per, n_per // 2), jnp.bfloat16),
            pltpu.VMEM((2 * (N_DEV - 1), m_per, n_per // 2), jnp.bfloat16),
            pltpu.SemaphoreType.DMA((2,)),
            pltpu.SemaphoreType.DMA((2 * (N_DEV - 1),)),
            pltpu.SemaphoreType.DMA((2 * (N_DEV - 1),)),
        ],
        compiler_params=pltpu.CompilerParams(
            collective_id=0, vmem_limit_bytes=100 * 1024 * 1024,
        ),
    )(x, w_mat)
